# baseline (speedup 1.0000x reference)
import functools

import jax
import jax.numpy as jnp
from jax import lax
from jax.experimental import pallas as pl
from jax.experimental.pallas import tpu as pltpu
from jax.experimental.pallas import tpu_sc as plsc

VOCAB = 1000000
DIM = 32
B = 16384
NEG = 20

NC = 2
NS = 16
NW = NC * NS
BPW = B // NW
NEG_PW = BPW * NEG
CH = 128
C_CH = BPW // CH
N_CH = NEG_PW // CH
N_GRP = 16

_mesh = plsc.VectorSubcoreMesh(core_axis_name="c", subcore_axis_name="s")


@functools.partial(
    pl.kernel,
    mesh=_mesh,
    compiler_params=pltpu.CompilerParams(use_tc_tiling_on_sc=False),
    out_type=[
        jax.ShapeDtypeStruct((B, DIM), jnp.float32),
        jax.ShapeDtypeStruct((B, DIM), jnp.float32),
        jax.ShapeDtypeStruct((B * NEG, DIM), jnp.float32),
    ],
    scratch_types=[
        pltpu.VMEM((C_CH, CH), jnp.int32),
        pltpu.VMEM((C_CH, CH), jnp.int32),
        pltpu.VMEM((N_CH, CH), jnp.int32),
        pltpu.VMEM((BPW, DIM), jnp.float32),
        pltpu.VMEM((BPW, DIM), jnp.float32),
        pltpu.VMEM((N_GRP * CH, DIM), jnp.float32),
        pltpu.SemaphoreType.DMA,
    ],
)
def _sc_gather(cw_hbm, xw_hbm, nw_hbm, ctab_hbm, xtab_hbm,
               cen_out, ctx_out, neg_out,
               cidx, xidx, nidx, cenbuf, ctxbuf, negbuf, sem):
    wid = lax.axis_index("s") * NC + lax.axis_index("c")
    base = wid * BPW

    pltpu.sync_copy(cw_hbm.at[wid], cidx)
    pltpu.sync_copy(xw_hbm.at[wid], xidx)
    pltpu.sync_copy(nw_hbm.at[wid], nidx)

    handles = []
    for j in range(C_CH):
        handles.append(pltpu.async_copy(
            ctab_hbm.at[cidx.at[j]], cenbuf.at[pl.ds(j * CH, CH)], sem))
    for j in range(C_CH):
        handles.append(pltpu.async_copy(
            xtab_hbm.at[xidx.at[j]], ctxbuf.at[pl.ds(j * CH, CH)], sem))
    for h in handles:
        h.wait()
    pltpu.sync_copy(cenbuf, cen_out.at[pl.ds(base, BPW)])
    pltpu.sync_copy(ctxbuf, ctx_out.at[pl.ds(base, BPW)])

    nbase = wid * NEG_PW
    for g in range(N_CH // N_GRP):
        handles = []
        for j in range(N_GRP):
            handles.append(pltpu.async_copy(
                xtab_hbm.at[nidx.at[g * N_GRP + j]],
                negbuf.at[pl.ds(j * CH, CH)], sem))
        for h in handles:
            h.wait()
        pltpu.sync_copy(
            negbuf, neg_out.at[pl.ds(nbase + g * N_GRP * CH, N_GRP * CH)])


_CHUNK = 1024


def _loss_body(cen_ref, ctx_ref, neg_ref, out_ref):
    cen = cen_ref[...]
    ctx = ctx_ref[...]
    neg = neg_ref[...].reshape(_CHUNK, NEG, DIM)
    pos_score = jnp.sum(cen * ctx, axis=1, keepdims=True)
    neg_score = jnp.sum(neg * cen[:, None, :], axis=2)
    pos_loss = jnp.log(1.0 / (1.0 + jnp.exp(-pos_score)) + 1e-10)
    neg_loss = jnp.log(1.0 / (1.0 + jnp.exp(neg_score)) + 1e-10)
    block = -(jnp.sum(pos_loss) + jnp.sum(neg_loss))

    @pl.when(pl.program_id(0) == 0)
    def _():
        out_ref[0, 0] = 0.0

    out_ref[0, 0] += block


def _tc_loss(cen, ctx, neg):
    nsteps = B // _CHUNK
    out = pl.pallas_call(
        _loss_body,
        grid=(nsteps,),
        in_specs=[
            pl.BlockSpec((_CHUNK, DIM), lambda i: (i, 0)),
            pl.BlockSpec((_CHUNK, DIM), lambda i: (i, 0)),
            pl.BlockSpec((_CHUNK * NEG, DIM), lambda i: (i, 0)),
        ],
        out_specs=pl.BlockSpec(memory_space=pltpu.SMEM),
        out_shape=jax.ShapeDtypeStruct((1, 1), jnp.float32),
    )(cen, ctx, neg)
    return out[0, 0] / jnp.float32(B)


def kernel(center_words, context_words, negative_words, center_table,
           context_table):
    cw = center_words.astype(jnp.int32).reshape(NW, C_CH, CH)
    xw = context_words.astype(jnp.int32).reshape(NW, C_CH, CH)
    nw = negative_words.astype(jnp.int32).reshape(NW, N_CH, CH)
    cen, ctx, neg = _sc_gather(cw, xw, nw, center_table, context_table)
    return _tc_loss(cen, ctx, neg)


# speedup vs baseline: 1.0280x; 1.0280x over previous
import functools

import jax
import jax.numpy as jnp
from jax import lax
from jax.experimental import pallas as pl
from jax.experimental.pallas import tpu as pltpu
from jax.experimental.pallas import tpu_sc as plsc

VOCAB = 1000000
DIM = 32
B = 16384
NEG = 20

NC = 2
NS = 16
L = 16
NW = NC * NS
BPW = B // NW
NEG_PW = BPW * NEG
CH = 128
C_CH = BPW // CH
N_CH = NEG_PW // CH
NBCH = 64
NSTR = NBCH * NEG // CH
NGRP = BPW // NBCH
BLKS = NBCH // L

_LN2 = 0.6931471805599453
_SQRT2 = 1.4142135623730951

_mesh = plsc.VectorSubcoreMesh(core_axis_name="c", subcore_axis_name="s")


def _vlog(x):
    bits = plsc.bitcast(x, jnp.int32)
    e = jnp.right_shift(bits, 23) - 127
    m = plsc.bitcast(
        jnp.bitwise_or(jnp.bitwise_and(bits, 0x007FFFFF), 0x3F800000),
        jnp.float32)
    big = m > _SQRT2
    m = jnp.where(big, m * 0.5, m)
    e = jnp.where(big, e + 1, e)
    z = (m - 1.0) / (m + 1.0)
    z2 = z * z
    p = 1.0 + z2 * (1.0 / 3.0 + z2 * (0.2 + z2 * (1.0 / 7.0 + z2 / 9.0)))
    return e.astype(jnp.float32) * _LN2 + 2.0 * z * p


def _log_sigmoid_eps(s):
    return _vlog(1.0 / (1.0 + jnp.exp(-s)) + 1e-10)


@functools.partial(
    pl.kernel,
    mesh=_mesh,
    compiler_params=pltpu.CompilerParams(
        use_tc_tiling_on_sc=False, needs_layout_passes=False),
    out_type=jax.ShapeDtypeStruct((NW, L), jnp.float32),
    scratch_types=[
        pltpu.VMEM((C_CH, CH), jnp.int32),
        pltpu.VMEM((C_CH, CH), jnp.int32),
        pltpu.VMEM((N_CH, CH), jnp.int32),
        pltpu.VMEM((BPW, DIM), jnp.float32),
        pltpu.VMEM((BPW, DIM), jnp.float32),
        pltpu.VMEM((NBCH * NEG, DIM), jnp.float32),
        pltpu.VMEM((L,), jnp.float32),
        pltpu.SemaphoreType.DMA,
    ],
)
def _sc_loss(cw_hbm, xw_hbm, nw_hbm, ctab_hbm, xtab_hbm, out_hbm,
             cidx, xidx, nidx, cen, ctx, negb, part, sem):
    wid = lax.axis_index("s") * NC + lax.axis_index("c")

    pltpu.sync_copy(cw_hbm.at[wid], cidx)
    pltpu.sync_copy(xw_hbm.at[wid], xidx)
    pltpu.sync_copy(nw_hbm.at[wid], nidx)

    handles = []
    for j in range(C_CH):
        handles.append(pltpu.async_copy(
            ctab_hbm.at[cidx.at[j]], cen.at[pl.ds(j * CH, CH)], sem))
        handles.append(pltpu.async_copy(
            xtab_hbm.at[xidx.at[j]], ctx.at[pl.ds(j * CH, CH)], sem))
    for h in handles:
        h.wait()

    part[...] = jnp.zeros((L,), jnp.float32)
    lanes = lax.iota(jnp.int32, L)

    def group_body(g, _):
        hs = []
        for j in range(NSTR):
            hs.append(pltpu.async_copy(
                xtab_hbm.at[nidx.at[g * NSTR + j]],
                negb.at[pl.ds(j * CH, CH)], sem))
        for h in hs:
            h.wait()

        def block_body(blk, _):
            rows = g * NBCH + blk * L + lanes
            nrow0 = (blk * L + lanes) * NEG
            acc_pos = jnp.zeros((L,), jnp.float32)
            acc_neg = [jnp.zeros((L,), jnp.float32) for _ in range(NEG)]
            for d in range(DIM):
                dcol = jnp.full((L,), d, jnp.int32)
                cvec = plsc.load_gather(cen, [rows, dcol])
                xvec = plsc.load_gather(ctx, [rows, dcol])
                acc_pos = acc_pos + cvec * xvec
                for k in range(NEG):
                    nvec = plsc.load_gather(negb, [nrow0 + k, dcol])
                    acc_neg[k] = acc_neg[k] + nvec * cvec
            total = _log_sigmoid_eps(acc_pos)
            for k in range(NEG):
                total = total + _log_sigmoid_eps(-acc_neg[k])
            part[...] = part[...] + total
            return 0

        lax.fori_loop(0, BLKS, block_body, 0)
        return 0

    lax.fori_loop(0, NGRP, group_body, 0)
    pltpu.sync_copy(part, out_hbm.at[wid])


def kernel(center_words, context_words, negative_words, center_table,
           context_table):
    cw = center_words.astype(jnp.int32).reshape(NW, C_CH, CH)
    xw = context_words.astype(jnp.int32).reshape(NW, C_CH, CH)
    nw = negative_words.astype(jnp.int32).reshape(NW, N_CH, CH)
    partials = _sc_loss(cw, xw, nw, center_table, context_table)
    return -jnp.sum(partials) / jnp.float32(B)


# speedup vs baseline: 1.1588x; 1.1273x over previous
import functools

import jax
import jax.numpy as jnp
from jax import lax
from jax.experimental import pallas as pl
from jax.experimental.pallas import tpu as pltpu
from jax.experimental.pallas import tpu_sc as plsc

VOCAB = 1000000
DIM = 32
B = 16384
NEG = 20

NC = 2
NS = 16
L = 16
NW = NC * NS
BPW = B // NW
NEG_PW = BPW * NEG
CH = 128
C_CH = BPW // CH
N_CH = NEG_PW // CH
NBCH = 64
NSTR = NBCH * NEG // CH
NGRP = BPW // NBCH
BLKS = NBCH // L

_LN2 = 0.6931471805599453
_SQRT2 = 1.4142135623730951

_mesh = plsc.VectorSubcoreMesh(core_axis_name="c", subcore_axis_name="s")


def _vlog(x):
    bits = plsc.bitcast(x, jnp.int32)
    e = jnp.right_shift(bits, 23) - 127
    m = plsc.bitcast(
        jnp.bitwise_or(jnp.bitwise_and(bits, 0x007FFFFF), 0x3F800000),
        jnp.float32)
    big = m > _SQRT2
    m = jnp.where(big, m * 0.5, m)
    e = jnp.where(big, e + 1, e)
    z = (m - 1.0) / (m + 1.0)
    z2 = z * z
    p = 1.0 + z2 * (1.0 / 3.0 + z2 * (0.2 + z2 * (1.0 / 7.0 + z2 / 9.0)))
    return e.astype(jnp.float32) * _LN2 + 2.0 * z * p


def _log_sigmoid_eps(s):
    return _vlog(1.0 / (1.0 + jnp.exp(-s)) + 1e-10)


@functools.partial(
    pl.kernel,
    mesh=_mesh,
    compiler_params=pltpu.CompilerParams(
        use_tc_tiling_on_sc=False, needs_layout_passes=False),
    out_type=jax.ShapeDtypeStruct((NW, L), jnp.float32),
    scratch_types=[
        pltpu.VMEM((BPW,), jnp.int32),
        pltpu.VMEM((BPW,), jnp.int32),
        pltpu.VMEM((BPW, NEG), jnp.int32),
        pltpu.VMEM((N_CH, CH), jnp.int32),
        pltpu.VMEM((BPW, DIM), jnp.float32),
        pltpu.VMEM((BPW, DIM), jnp.float32),
        pltpu.VMEM((NBCH * NEG, DIM), jnp.float32),
        pltpu.VMEM((L,), jnp.float32),
        pltpu.SemaphoreType.DMA,
    ],
)
def _sc_loss(cw_hbm, xw_hbm, nw_hbm, ctab_hbm, xtab_hbm, out_hbm,
             cidx, xidx, nraw, nidx, cen, ctx, negb, part, sem):
    wid = lax.axis_index("s") * NC + lax.axis_index("c")
    base = wid * BPW

    pltpu.sync_copy(cw_hbm.at[pl.ds(base, BPW)], cidx)
    pltpu.sync_copy(xw_hbm.at[pl.ds(base, BPW)], xidx)
    pltpu.sync_copy(nw_hbm.at[pl.ds(base, BPW), :], nraw)

    handles = []
    for j in range(C_CH):
        handles.append(pltpu.async_copy(
            ctab_hbm.at[cidx.at[pl.ds(j * CH, CH)]],
            cen.at[pl.ds(j * CH, CH)], sem))
        handles.append(pltpu.async_copy(
            xtab_hbm.at[xidx.at[pl.ds(j * CH, CH)]],
            ctx.at[pl.ds(j * CH, CH)], sem))

    lanes = lax.iota(jnp.int32, L)

    def repack_body(i, _):
        p = lanes + i * L
        g = plsc.load_gather(
            nraw, [lax.div(p, NEG), lax.rem(p, NEG)])
        nidx[lax.div(i, CH // L), pl.ds(lax.rem(i, CH // L) * L, L)] = g
        return 0

    lax.fori_loop(0, NEG_PW // L, repack_body, 0)

    for h in handles:
        h.wait()

    part[...] = jnp.zeros((L,), jnp.float32)

    def group_body(g, _):
        hs = []
        for j in range(NSTR):
            hs.append(pltpu.async_copy(
                xtab_hbm.at[nidx.at[g * NSTR + j]],
                negb.at[pl.ds(j * CH, CH)], sem))
        for h in hs:
            h.wait()

        def block_body(blk, _):
            rows = g * NBCH + blk * L + lanes
            nrow0 = (blk * L + lanes) * NEG
            acc_pos = jnp.zeros((L,), jnp.float32)
            acc_neg = [jnp.zeros((L,), jnp.float32) for _ in range(NEG)]
            for d in range(DIM):
                dcol = jnp.bitwise_and(lanes + d, DIM - 1)
                cvec = plsc.load_gather(cen, [rows, dcol])
                xvec = plsc.load_gather(ctx, [rows, dcol])
                acc_pos = acc_pos + cvec * xvec
                for k in range(NEG):
                    nvec = plsc.load_gather(negb, [nrow0 + k, dcol])
                    acc_neg[k] = acc_neg[k] + nvec * cvec
            total = _log_sigmoid_eps(acc_pos)
            for k in range(NEG):
                total = total + _log_sigmoid_eps(-acc_neg[k])
            part[...] = part[...] + total
            return 0

        lax.fori_loop(0, BLKS, block_body, 0)
        return 0

    lax.fori_loop(0, NGRP, group_body, 0)
    pltpu.sync_copy(part, out_hbm.at[wid])


def kernel(center_words, context_words, negative_words, center_table,
           context_table):
    partials = _sc_loss(center_words.astype(jnp.int32),
                        context_words.astype(jnp.int32),
                        negative_words.astype(jnp.int32),
                        center_table, context_table)
    return -jnp.sum(partials) / jnp.float32(B)


# speedup vs baseline: 1.1630x; 1.0036x over previous
import functools

import jax
import jax.numpy as jnp
from jax import lax
from jax.experimental import pallas as pl
from jax.experimental.pallas import tpu as pltpu
from jax.experimental.pallas import tpu_sc as plsc

VOCAB = 1000000
DIM = 32
B = 16384
NEG = 20

NC = 2
NS = 16
L = 16
NW = NC * NS
BPW = B // NW
NEG_PW = BPW * NEG
CH = 128
C_CH = BPW // CH
N_CH = NEG_PW // CH
NBCH = 64
NSTR = NBCH * NEG // CH
NGRP = BPW // NBCH
BLKS = NBCH // L

_LN2 = 0.6931471805599453
_SQRT2 = 1.4142135623730951

_mesh = plsc.VectorSubcoreMesh(core_axis_name="c", subcore_axis_name="s")


def _vlog(x):
    bits = plsc.bitcast(x, jnp.int32)
    e = jnp.right_shift(bits, 23) - 127
    m = plsc.bitcast(
        jnp.bitwise_or(jnp.bitwise_and(bits, 0x007FFFFF), 0x3F800000),
        jnp.float32)
    big = m > _SQRT2
    m = jnp.where(big, m * 0.5, m)
    e = jnp.where(big, e + 1, e)
    z = (m - 1.0) / (m + 1.0)
    z2 = z * z
    p = 1.0 + z2 * (1.0 / 3.0 + z2 * (0.2 + z2 * (1.0 / 7.0 + z2 / 9.0)))
    return e.astype(jnp.float32) * _LN2 + 2.0 * z * p


def _log_sigmoid_eps(s):
    return _vlog(1.0 / (1.0 + jnp.exp(-s)) + 1e-10)


def _widen_body(nw_ref, out_ref):
    out_ref[...] = jnp.concatenate(
        [nw_ref[...], jnp.zeros((_WBLK, 128 - NEG), jnp.int32)], axis=1)


_WBLK = 1024


def _widen_neg(nw):
    return pl.pallas_call(
        _widen_body,
        grid=(B // _WBLK,),
        in_specs=[pl.BlockSpec((_WBLK, NEG), lambda i: (i, 0))],
        out_specs=pl.BlockSpec((_WBLK, 128), lambda i: (i, 0)),
        out_shape=jax.ShapeDtypeStruct((B, 128), jnp.int32),
    )(nw)


@functools.partial(
    pl.kernel,
    mesh=_mesh,
    compiler_params=pltpu.CompilerParams(
        use_tc_tiling_on_sc=False, needs_layout_passes=False),
    out_type=jax.ShapeDtypeStruct((NW, L), jnp.float32),
    scratch_types=[
        pltpu.VMEM((BPW,), jnp.int32),
        pltpu.VMEM((BPW,), jnp.int32),
        pltpu.VMEM((BPW, 24), jnp.int32),
        pltpu.VMEM((N_CH, CH), jnp.int32),
        pltpu.VMEM((BPW, DIM), jnp.float32),
        pltpu.VMEM((BPW, DIM), jnp.float32),
        pltpu.VMEM((NBCH * NEG, DIM), jnp.float32),
        pltpu.VMEM((L,), jnp.float32),
        pltpu.SemaphoreType.DMA,
    ],
)
def _sc_loss(cw_hbm, xw_hbm, nw_hbm, ctab_hbm, xtab_hbm, out_hbm,
             cidx, xidx, nraw, nidx, cen, ctx, negb, part, sem):
    wid = lax.axis_index("s") * NC + lax.axis_index("c")
    base = wid * BPW

    pltpu.sync_copy(cw_hbm.at[pl.ds(base, BPW)], cidx)
    pltpu.sync_copy(xw_hbm.at[pl.ds(base, BPW)], xidx)
    pltpu.sync_copy(nw_hbm.at[pl.ds(base, BPW), pl.ds(0, 24)], nraw)

    handles = []
    for j in range(C_CH):
        handles.append(pltpu.async_copy(
            ctab_hbm.at[cidx.at[pl.ds(j * CH, CH)]],
            cen.at[pl.ds(j * CH, CH)], sem))
        handles.append(pltpu.async_copy(
            xtab_hbm.at[xidx.at[pl.ds(j * CH, CH)]],
            ctx.at[pl.ds(j * CH, CH)], sem))

    lanes = lax.iota(jnp.int32, L)

    def repack_body(i, _):
        p = lanes + i * L
        g = plsc.load_gather(
            nraw, [lax.div(p, NEG), lax.rem(p, NEG)])
        nidx[lax.div(i, CH // L), pl.ds(lax.rem(i, CH // L) * L, L)] = g
        return 0

    lax.fori_loop(0, NEG_PW // L, repack_body, 0)

    for h in handles:
        h.wait()

    part[...] = jnp.zeros((L,), jnp.float32)

    def group_body(g, _):
        hs = []
        for j in range(NSTR):
            hs.append(pltpu.async_copy(
                xtab_hbm.at[nidx.at[g * NSTR + j]],
                negb.at[pl.ds(j * CH, CH)], sem))
        for h in hs:
            h.wait()

        def block_body(blk, _):
            rows = g * NBCH + blk * L + lanes
            nrow0 = (blk * L + lanes) * NEG
            acc_pos = jnp.zeros((L,), jnp.float32)
            acc_neg = [jnp.zeros((L,), jnp.float32) for _ in range(NEG)]
            for d in range(DIM):
                dcol = jnp.bitwise_and(lanes + d, DIM - 1)
                cvec = plsc.load_gather(cen, [rows, dcol])
                xvec = plsc.load_gather(ctx, [rows, dcol])
                acc_pos = acc_pos + cvec * xvec
                for k in range(NEG):
                    nvec = plsc.load_gather(negb, [nrow0 + k, dcol])
                    acc_neg[k] = acc_neg[k] + nvec * cvec
            total = _log_sigmoid_eps(acc_pos)
            for k in range(NEG):
                total = total + _log_sigmoid_eps(-acc_neg[k])
            part[...] = part[...] + total
            return 0

        lax.fori_loop(0, BLKS, block_body, 0)
        return 0

    lax.fori_loop(0, NGRP, group_body, 0)
    pltpu.sync_copy(part, out_hbm.at[wid])


def kernel(center_words, context_words, negative_words, center_table,
           context_table):
    nw128 = _widen_neg(negative_words.astype(jnp.int32))
    partials = _sc_loss(center_words.astype(jnp.int32),
                        context_words.astype(jnp.int32),
                        nw128, center_table, context_table)
    return -jnp.sum(partials) / jnp.float32(B)
